# baseline (speedup 1.0000x reference)
import functools

import jax
import jax.numpy as jnp
from jax import lax
from jax.experimental import pallas as pl
from jax.experimental.pallas import tpu as pltpu
from jax.experimental.pallas import tpu_sc as plsc

D = 32
L = 16
NC = 2
NS = 16
NW = NC * NS


def _gmf_body(user_hbm, item_hbm, ut_hbm, it_hbm, wb_hbm, bb_hbm, out_hbm,
              uidx, iidx, urows, irows, outv, wv, bv, sem_u, sem_i, bpw):
    wid = lax.axis_index("s") * NC + lax.axis_index("c")
    base = wid * bpw

    pltpu.sync_copy(user_hbm.at[pl.ds(base, bpw)], uidx)
    pltpu.sync_copy(item_hbm.at[pl.ds(base, bpw)], iidx)
    cp_u = pltpu.async_copy(ut_hbm.at[uidx], urows, sem_u)
    cp_i = pltpu.async_copy(it_hbm.at[iidx], irows, sem_i)

    pltpu.sync_copy(wb_hbm, wv)
    pltpu.sync_copy(bb_hbm, bv)

    cp_u.wait()
    cp_i.wait()

    wvecs = [wv[d] for d in range(D)]
    bvec = bv[...]
    lane = lax.iota(jnp.int32, L)
    cols = [jnp.full((L,), d, jnp.int32) for d in range(D)]

    def group(g, _):
        rows = g * L + lane
        acc = bvec
        for d in range(D):
            uc = plsc.load_gather(urows, [rows, cols[d]])
            ic = plsc.load_gather(irows, [rows, cols[d]])
            acc = acc + uc * ic * wvecs[d]
        outv[pl.ds(g * L, L)] = acc
        return 0

    lax.fori_loop(0, bpw // L, group, 0)

    pltpu.sync_copy(outv, out_hbm.at[pl.ds(base, bpw)])


def kernel(user, item, user_table, item_table, W, b):
    batch = user.shape[0]
    bpw = batch // NW
    mesh = plsc.VectorSubcoreMesh(core_axis_name="c", subcore_axis_name="s")

    wb = jnp.broadcast_to(W.reshape(D, 1), (D, L)).astype(jnp.float32)
    bb = jnp.broadcast_to(b.reshape(1), (L,)).astype(jnp.float32)

    k = functools.partial(
        pl.kernel,
        mesh=mesh,
        out_type=jax.ShapeDtypeStruct((batch,), jnp.float32),
        scratch_types=[
            pltpu.VMEM((bpw,), jnp.int32),
            pltpu.VMEM((bpw,), jnp.int32),
            pltpu.VMEM((bpw, D), jnp.float32),
            pltpu.VMEM((bpw, D), jnp.float32),
            pltpu.VMEM((bpw,), jnp.float32),
            pltpu.VMEM((D, L), jnp.float32),
            pltpu.VMEM((L,), jnp.float32),
            pltpu.SemaphoreType.DMA,
            pltpu.SemaphoreType.DMA,
        ],
        compiler_params=pltpu.CompilerParams(
            needs_layout_passes=False, use_tc_tiling_on_sc=False),
    )(functools.partial(_gmf_body, bpw=bpw))

    return k(user.astype(jnp.int32), item.astype(jnp.int32),
             user_table, item_table, wb, bb)
